# baseline (speedup 1.0000x reference)
import functools

import jax
import jax.numpy as jnp
from jax import lax
from jax.experimental import pallas as pl
from jax.experimental.pallas import tpu as pltpu
from jax.experimental.pallas import tpu_sc as plsc

NC = 2
NS = 16
NW = NC * NS

CHUNK = 128
NBUF = 8
LAG = 4


@functools.partial(jax.jit, static_argnames=("n_idx", "embed"))
def _sc_gather(idx_flat, table, *, n_idx, embed):
    n_per_w = n_idx // NW
    n_chunks = n_per_w // CHUNK
    n_groups = n_chunks // NBUF
    idx_3d = idx_flat.reshape(NW, n_chunks, CHUNK)

    mesh = plsc.VectorSubcoreMesh(
        core_axis_name="c", subcore_axis_name="s",
        num_cores=NC, num_subcores=NS)

    @functools.partial(
        pl.kernel,
        out_type=jax.ShapeDtypeStruct((n_idx, embed), jnp.float32),
        mesh=mesh,
        scratch_types=[
            pltpu.VMEM((n_chunks, CHUNK), jnp.int32),
            pltpu.VMEM((NBUF, CHUNK, embed), jnp.float32),
            pltpu.SemaphoreType.DMA((NBUF,)),
            pltpu.SemaphoreType.DMA((NBUF,)),
        ],
        compiler_params=pltpu.CompilerParams(use_tc_tiling_on_sc=False),
    )
    def body(idx_hbm, table_hbm, out_hbm, idx_v, rows_v, gsems, wsems):
        wid = lax.axis_index("s") * NC + lax.axis_index("c")
        base = wid * n_per_w
        pltpu.sync_copy(idx_hbm.at[wid], idx_v)

        def gather(j, b):
            return pltpu.make_async_copy(
                table_hbm.at[idx_v.at[j]], rows_v.at[b], gsems.at[b])

        def write(j, b):
            return pltpu.make_async_copy(
                rows_v.at[b],
                out_hbm.at[pl.ds(base + j * CHUNK, CHUNK)],
                wsems.at[b])

        for b in range(LAG):
            gather(b, b).start()

        def group(g, carry):
            for b in range(NBUF):
                j = g * NBUF + b
                bg = (b + LAG) % NBUF
                jg = j + LAG
                @pl.when(jg >= NBUF)
                def _():
                    write(jg - NBUF, bg).wait()
                @pl.when(jg < n_chunks)
                def _():
                    gather(jg, bg).start()
                gather(j, b).wait()
                write(j, b).start()
            return carry

        lax.fori_loop(0, n_groups, group, 0)
        for i in range(NBUF - LAG):
            j = n_chunks - (NBUF - LAG) + i
            write(j, j % NBUF).wait()

    return body(idx_3d, table)


def kernel(inputs, table):
    bsz, seq = inputs.shape
    vocab, embed = table.shape
    n_idx = bsz * seq
    out = _sc_gather(inputs.reshape(n_idx), table, n_idx=n_idx, embed=embed)
    return out.reshape(bsz, seq, embed)


# speedup vs baseline: 1.3278x; 1.3278x over previous
import functools

import jax
import jax.numpy as jnp
from jax import lax
from jax.experimental import pallas as pl
from jax.experimental.pallas import tpu as pltpu
from jax.experimental.pallas import tpu_sc as plsc

NC = 2
NS = 16
NW = NC * NS

CHUNK = 128
NBUF = 8
LAG = 4


@functools.partial(jax.jit, static_argnames=("n_idx", "embed"))
def _sc_gather(idx_flat, table, *, n_idx, embed):
    n_per_w = n_idx // NW
    n_chunks = n_per_w // CHUNK
    n_groups = n_chunks // NBUF
    idx_3d = idx_flat.reshape(NW, n_chunks, CHUNK)

    mesh = plsc.VectorSubcoreMesh(
        core_axis_name="c", subcore_axis_name="s",
        num_cores=NC, num_subcores=NS)

    @functools.partial(
        pl.kernel,
        out_type=jax.ShapeDtypeStruct((n_idx, 2 * embed), jnp.float32),
        mesh=mesh,
        scratch_types=[
            pltpu.VMEM((n_chunks, CHUNK), jnp.int32),
            pltpu.VMEM((NBUF, CHUNK, embed), jnp.float32),
            pltpu.SemaphoreType.DMA((NBUF,)),
            pltpu.SemaphoreType.DMA((NBUF,)),
        ],
        compiler_params=pltpu.CompilerParams(use_tc_tiling_on_sc=False),
    )
    def body(idx_hbm, table_hbm, out_hbm, idx_v, rows_v, gsems, wsems):
        wid = lax.axis_index("s") * NC + lax.axis_index("c")
        base = wid * n_per_w
        pltpu.sync_copy(idx_hbm.at[wid], idx_v)

        def gather(j, b):
            return pltpu.make_async_copy(
                table_hbm.at[idx_v.at[j]], rows_v.at[b], gsems.at[b])

        def write(j, b):
            return pltpu.make_async_copy(
                rows_v.at[b],
                out_hbm.at[pl.ds(base + j * CHUNK, CHUNK), pl.ds(0, embed)],
                wsems.at[b])

        for b in range(LAG):
            gather(b, b).start()

        def group(g, carry):
            for b in range(NBUF):
                j = g * NBUF + b
                bg = (b + LAG) % NBUF
                jg = j + LAG
                @pl.when(jg >= NBUF)
                def _():
                    write(jg - NBUF, bg).wait()
                @pl.when(jg < n_chunks)
                def _():
                    gather(jg, bg).start()
                gather(j, b).wait()
                write(j, b).start()
            return carry

        lax.fori_loop(0, n_groups, group, 0)
        for i in range(NBUF - LAG):
            j = n_chunks - (NBUF - LAG) + i
            write(j, j % NBUF).wait()

    return body(idx_3d, table)


def kernel(inputs, table):
    bsz, seq = inputs.shape
    vocab, embed = table.shape
    n_idx = bsz * seq
    out = _sc_gather(inputs.reshape(n_idx), table, n_idx=n_idx, embed=embed)
    return out[:, :embed].reshape(bsz, seq, embed)


# speedup vs baseline: 1.3298x; 1.0015x over previous
import functools

import jax
import jax.numpy as jnp
from jax import lax
from jax.experimental import pallas as pl
from jax.experimental.pallas import tpu as pltpu
from jax.experimental.pallas import tpu_sc as plsc

NC = 2
NS = 16
NW = NC * NS

CHUNK = 256
NBUF = 4
LAG = 2


@functools.partial(jax.jit, static_argnames=("n_idx", "embed"))
def _sc_gather(idx_flat, table, *, n_idx, embed):
    n_per_w = n_idx // NW
    n_chunks = n_per_w // CHUNK
    n_groups = n_chunks // NBUF
    idx_3d = idx_flat.reshape(NW, n_chunks, CHUNK)

    mesh = plsc.VectorSubcoreMesh(
        core_axis_name="c", subcore_axis_name="s",
        num_cores=NC, num_subcores=NS)

    @functools.partial(
        pl.kernel,
        out_type=jax.ShapeDtypeStruct((n_idx, 2 * embed), jnp.float32),
        mesh=mesh,
        scratch_types=[
            pltpu.VMEM((n_chunks, CHUNK), jnp.int32),
            pltpu.VMEM((NBUF, CHUNK, embed), jnp.float32),
            pltpu.SemaphoreType.DMA((NBUF,)),
            pltpu.SemaphoreType.DMA((NBUF,)),
        ],
        compiler_params=pltpu.CompilerParams(use_tc_tiling_on_sc=False),
    )
    def body(idx_hbm, table_hbm, out_hbm, idx_v, rows_v, gsems, wsems):
        wid = lax.axis_index("s") * NC + lax.axis_index("c")
        base = wid * n_per_w
        pltpu.sync_copy(idx_hbm.at[wid], idx_v)

        def gather(j, b):
            return pltpu.make_async_copy(
                table_hbm.at[idx_v.at[j]], rows_v.at[b], gsems.at[b])

        def write(j, b):
            return pltpu.make_async_copy(
                rows_v.at[b],
                out_hbm.at[pl.ds(base + j * CHUNK, CHUNK), pl.ds(0, embed)],
                wsems.at[b])

        for b in range(LAG):
            gather(b, b).start()

        def group(g, carry):
            for b in range(NBUF):
                j = g * NBUF + b
                bg = (b + LAG) % NBUF
                jg = j + LAG
                @pl.when(jg >= NBUF)
                def _():
                    write(jg - NBUF, bg).wait()
                @pl.when(jg < n_chunks)
                def _():
                    gather(jg, bg).start()
                gather(j, b).wait()
                write(j, b).start()
            return carry

        lax.fori_loop(0, n_groups, group, 0)
        for i in range(NBUF - LAG):
            j = n_chunks - (NBUF - LAG) + i
            write(j, j % NBUF).wait()

    return body(idx_3d, table)


def kernel(inputs, table):
    bsz, seq = inputs.shape
    vocab, embed = table.shape
    n_idx = bsz * seq
    out = _sc_gather(inputs.reshape(n_idx), table, n_idx=n_idx, embed=embed)
    return out[:, :embed].reshape(bsz, seq, embed)
